# baseline (speedup 1.0000x reference)
import jax
import jax.numpy as jnp
from jax import lax
from jax.experimental import pallas as pl
from jax.experimental.pallas import tpu as pltpu
from jax.experimental.pallas import tpu_sc as plsc

NC, NS = 2, 16
NW = NC * NS
B = 16384
D = 128
BPW = B // NW
C = 5
R = 64
NCHUNK = BPW // R
NV = 16
WBN = C * 2 * D + NV


def _fused_body(user_t, game_t, user_idx, game_idx, wb_h, out_h,
                idxu, idxg, ub0, ub1, gb0, gb1, wv, outv,
                su0, su1, sg0, sg1):
    wid = lax.axis_index("s") * NC + lax.axis_index("c")
    base = wid * BPW
    pltpu.sync_copy(user_idx.at[pl.ds(base, BPW)], idxu)
    pltpu.sync_copy(game_idx.at[pl.ds(base, BPW)], idxg)
    pltpu.sync_copy(wb_h, wv)

    ubufs, gbufs = (ub0, ub1), (gb0, gb1)
    usems, gsems = (su0, su1), (sg0, sg1)

    def start(k):
        s = k % 2
        cu = pltpu.async_copy(user_t.at[idxu.at[pl.ds(k * R, R)]],
                              ubufs[s], usems[s])
        cg = pltpu.async_copy(game_t.at[idxg.at[pl.ds(k * R, R)]],
                              gbufs[s], gsems[s])
        return cu, cg

    pend = start(0)
    for k in range(NCHUNK):
        nxt = start(k + 1) if k + 1 < NCHUNK else None
        pend[0].wait()
        pend[1].wait()
        ub, gb = ubufs[k % 2], gbufs[k % 2]
        lanes = lax.iota(jnp.int32, NV)
        bias_vec = wv[pl.ds(C * 2 * D, NV)]

        for grp in ((0, 1), (2, 3), (4,)):
            wregs = [[wv[pl.ds(c * 2 * D + NV * j, NV)] for j in range(16)]
                     for c in grp]
            cmasks = [lanes == c for c in grp]
            gmask = (lanes >= grp[0]) & (lanes <= grp[-1])

            def bbody(b, carry, wregs=wregs, cmasks=cmasks, gmask=gmask,
                      grp=grp, ub=ub, gb=gb, k=k):
                rows = ([ub[b, pl.ds(NV * j, NV)] for j in range(8)]
                        + [gb[b, pl.ds(NV * j, NV)] for j in range(8)])
                v = bias_vec
                for ci, c in enumerate(grp):
                    acc = [rows[j] * wregs[ci][j] for j in range(16)]
                    while len(acc) > 1:
                        acc = [acc[i] + acc[i + 1]
                               for i in range(0, len(acc), 2)]
                    tot = plsc.cumsum(acc[0])[NV - 1]
                    v = jnp.where(cmasks[ci], bias_vec + tot, v)
                row_splat = jnp.broadcast_to(k * R + b, (NV,))
                plsc.store_scatter(outv, [row_splat, lanes], v, mask=gmask)
                return carry

            lax.fori_loop(0, R, bbody, 0, unroll=4)
        pend = nxt
    pltpu.sync_copy(outv, out_h.at[pl.ds(base, BPW)])


_sc_fused = pl.kernel(
    _fused_body,
    out_type=jax.ShapeDtypeStruct((B, C), jnp.float32),
    mesh=plsc.VectorSubcoreMesh(core_axis_name="c", subcore_axis_name="s"),
    compiler_params=pltpu.CompilerParams(needs_layout_passes=False),
    scratch_types=[
        pltpu.VMEM((BPW,), jnp.int32),
        pltpu.VMEM((BPW,), jnp.int32),
        pltpu.VMEM((R, D), jnp.float32),
        pltpu.VMEM((R, D), jnp.float32),
        pltpu.VMEM((R, D), jnp.float32),
        pltpu.VMEM((R, D), jnp.float32),
        pltpu.VMEM((WBN,), jnp.float32),
        pltpu.VMEM((BPW, C), jnp.float32),
        pltpu.SemaphoreType.DMA,
        pltpu.SemaphoreType.DMA,
        pltpu.SemaphoreType.DMA,
        pltpu.SemaphoreType.DMA,
    ],
)


def kernel(user, game, user_table, game_table, fc_w, fc_b):
    wb = jnp.concatenate([fc_w.reshape(-1), fc_b,
                          jnp.zeros((NV - C,), jnp.float32)])
    return _sc_fused(user_table, game_table, user, game, wb)


# speedup vs baseline: 1.2809x; 1.2809x over previous
import jax
import jax.numpy as jnp
from jax import lax
from jax.experimental import pallas as pl
from jax.experimental.pallas import tpu as pltpu
from jax.experimental.pallas import tpu_sc as plsc

NC, NS = 2, 16
NW = NC * NS
B = 16384
D = 128
BPW = B // NW
C = 5
R = 64
NCHUNK = BPW // R
NV = 16
WBN = C * 2 * D + NV


def _fused_body(user_t, game_t, user_idx, game_idx, wb_h, out_h,
                idxu, idxg, ub0, ub1, gb0, gb1, wv, outv,
                su0, su1, sg0, sg1):
    wid = lax.axis_index("s") * NC + lax.axis_index("c")
    base = wid * BPW
    pltpu.sync_copy(user_idx.at[pl.ds(base, BPW)], idxu)
    pltpu.sync_copy(game_idx.at[pl.ds(base, BPW)], idxg)
    pltpu.sync_copy(wb_h, wv)

    ubufs, gbufs = (ub0, ub1), (gb0, gb1)
    usems, gsems = (su0, su1), (sg0, sg1)

    def start(k):
        s = k % 2
        cu = pltpu.async_copy(user_t.at[idxu.at[pl.ds(k * R, R)]],
                              ubufs[s], usems[s])
        cg = pltpu.async_copy(game_t.at[idxg.at[pl.ds(k * R, R)]],
                              gbufs[s], gsems[s])
        return cu, cg

    pend = start(0)
    for k in range(NCHUNK):
        nxt = start(k + 1) if k + 1 < NCHUNK else None
        pend[0].wait()
        pend[1].wait()
        ub, gb = ubufs[k % 2], gbufs[k % 2]
        lanes = lax.iota(jnp.int32, NV)
        bias_vec = wv[pl.ds(C * 2 * D, NV)]

        for grp in ((0, 1), (2, 3), (4,)):
            wregs = [[wv[pl.ds(c * 2 * D + NV * j, NV)] for j in range(16)]
                     for c in grp]
            cmasks = [lanes == c for c in grp]
            gmask = (lanes >= grp[0]) & (lanes <= grp[-1])

            @plsc.parallel_loop(0, R, unroll=2)
            def bbody(b, wregs=wregs, cmasks=cmasks, gmask=gmask,
                      grp=grp, ub=ub, gb=gb, k=k):
                rows = ([ub[b, pl.ds(NV * j, NV)] for j in range(8)]
                        + [gb[b, pl.ds(NV * j, NV)] for j in range(8)])
                v = bias_vec
                for ci, c in enumerate(grp):
                    acc = [rows[j] * wregs[ci][j] for j in range(16)]
                    while len(acc) > 1:
                        acc = [acc[i] + acc[i + 1]
                               for i in range(0, len(acc), 2)]
                    tot = plsc.cumsum(acc[0])[NV - 1]
                    v = jnp.where(cmasks[ci], bias_vec + tot, v)
                row_splat = jnp.broadcast_to(k * R + b, (NV,))
                plsc.store_scatter(outv, [row_splat, lanes], v, mask=gmask)
        pend = nxt
    pltpu.sync_copy(outv, out_h.at[pl.ds(base, BPW)])


_sc_fused = pl.kernel(
    _fused_body,
    out_type=jax.ShapeDtypeStruct((B, C), jnp.float32),
    mesh=plsc.VectorSubcoreMesh(core_axis_name="c", subcore_axis_name="s"),
    compiler_params=pltpu.CompilerParams(needs_layout_passes=False),
    scratch_types=[
        pltpu.VMEM((BPW,), jnp.int32),
        pltpu.VMEM((BPW,), jnp.int32),
        pltpu.VMEM((R, D), jnp.float32),
        pltpu.VMEM((R, D), jnp.float32),
        pltpu.VMEM((R, D), jnp.float32),
        pltpu.VMEM((R, D), jnp.float32),
        pltpu.VMEM((WBN,), jnp.float32),
        pltpu.VMEM((BPW, C), jnp.float32),
        pltpu.SemaphoreType.DMA,
        pltpu.SemaphoreType.DMA,
        pltpu.SemaphoreType.DMA,
        pltpu.SemaphoreType.DMA,
    ],
)


def kernel(user, game, user_table, game_table, fc_w, fc_b):
    wb = jnp.concatenate([fc_w.reshape(-1), fc_b,
                          jnp.zeros((NV - C,), jnp.float32)])
    return _sc_fused(user_table, game_table, user, game, wb)
